# baseline (speedup 1.0000x reference)
import functools

import jax
import jax.numpy as jnp
from jax import lax
from jax.experimental import pallas as pl
from jax.experimental.pallas import tpu as pltpu
from jax.experimental.pallas import tpu_sc as plsc

BATCH = 4096
HIST = 200
EMBED_DIM = 64
OUTPUT_DIM = 128

NC = 2
NS = 16
NW = NC * NS

ROWS_PER_W = BATCH // NW
HALF = 104
REAL_HALF = 100


NBUF = 4


def _sc_pool_body(x_hbm, table_hbm, out_hbm, idx_v, ring_v, out_stage, *sems):
    wid = lax.axis_index("s") * NC + lax.axis_index("c")
    pltpu.sync_copy(x_hbm.at[pl.ds(wid * (2 * ROWS_PER_W), 2 * ROWS_PER_W)], idx_v)

    def slot_copies(s, r):
        base = s * (2 * HALF)
        return (
            pltpu.make_async_copy(
                table_hbm.at[idx_v.at[2 * r]],
                ring_v.at[pl.ds(base, HALF)],
                sems[s],
            ),
            pltpu.make_async_copy(
                table_hbm.at[idx_v.at[2 * r + 1]],
                ring_v.at[pl.ds(base + HALF, HALF)],
                sems[s],
            ),
        )

    def issue(s, r):
        for c in slot_copies(s, r):
            c.start()

    for s in range(NBUF):
        issue(s, s)

    def outer(g, carry):
        for s in range(NBUF):
            r = g * NBUF + s
            for c in slot_copies(s, r):
                c.wait()
            base = s * (2 * HALF)

            def red(i, accs):
                a0, a1, a2, a3 = accs
                j0 = base + i
                j1 = base + HALF + i
                a0 = a0 + ring_v[j0, pl.ds(0, 16)] + ring_v[j1, pl.ds(0, 16)]
                a1 = a1 + ring_v[j0, pl.ds(16, 16)] + ring_v[j1, pl.ds(16, 16)]
                a2 = a2 + ring_v[j0, pl.ds(32, 16)] + ring_v[j1, pl.ds(32, 16)]
                a3 = a3 + ring_v[j0, pl.ds(48, 16)] + ring_v[j1, pl.ds(48, 16)]
                return (a0, a1, a2, a3)

            z = jnp.zeros((16,), jnp.float32)
            a0, a1, a2, a3 = (
                ring_v[base, pl.ds(0, 16)],
                ring_v[base, pl.ds(16, 16)],
                ring_v[base, pl.ds(32, 16)],
                ring_v[base, pl.ds(48, 16)],
            )
            scale = jnp.float32(1.0 / HIST)
            out_stage[r, pl.ds(0, 16)] = a0 * scale
            out_stage[r, pl.ds(16, 16)] = a1 * scale
            out_stage[r, pl.ds(32, 16)] = a2 * scale
            out_stage[r, pl.ds(48, 16)] = a3 * scale

            @pl.when(r + NBUF < ROWS_PER_W)
            def _():
                issue(s, r + NBUF)

        return carry

    lax.fori_loop(0, ROWS_PER_W // NBUF, outer, 0)
    pltpu.sync_copy(out_stage, out_hbm.at[pl.ds(wid * ROWS_PER_W, ROWS_PER_W)])


_sc_pool = functools.partial(
    pl.kernel,
    out_type=jax.ShapeDtypeStruct((BATCH, EMBED_DIM), jnp.float32),
    mesh=plsc.VectorSubcoreMesh(core_axis_name="c", subcore_axis_name="s"),
    scratch_types=[
        pltpu.VMEM((2 * ROWS_PER_W, HALF), jnp.int32),
        pltpu.VMEM((NBUF * 2 * HALF, EMBED_DIM), jnp.float32),
        pltpu.VMEM((ROWS_PER_W, EMBED_DIM), jnp.float32),
    ]
    + [pltpu.SemaphoreType.DMA] * NBUF,
    compiler_params=pltpu.CompilerParams(use_tc_tiling_on_sc=False),
)(_sc_pool_body)


def _tc_head_body(p_ref, w_ref, b_ref, o_ref):
    out = jnp.dot(p_ref[...], w_ref[...], preferred_element_type=jnp.float32)
    out = out + b_ref[...]
    ss = jnp.sum(out * out, axis=1, keepdims=True)
    norm = jnp.sqrt(ss)
    o_ref[...] = out / jnp.maximum(norm, 1e-12)


def _tc_head(pooled, wt, b2):
    blk = 512
    return pl.pallas_call(
        _tc_head_body,
        grid=(BATCH // blk,),
        in_specs=[
            pl.BlockSpec((blk, EMBED_DIM), lambda i: (i, 0)),
            pl.BlockSpec((EMBED_DIM, OUTPUT_DIM), lambda i: (0, 0)),
            pl.BlockSpec((1, OUTPUT_DIM), lambda i: (0, 0)),
        ],
        out_specs=pl.BlockSpec((blk, OUTPUT_DIM), lambda i: (i, 0)),
        out_shape=jax.ShapeDtypeStruct((BATCH, OUTPUT_DIM), jnp.float32),
    )(pooled, wt, b2)


def kernel(x, table, W, b):
    x = x.astype(jnp.int32)
    x3 = x.reshape(BATCH, 2, REAL_HALF)
    x3 = jnp.pad(x3, ((0, 0), (0, 0), (0, HALF - REAL_HALF)))
    x3 = x3.reshape(2 * BATCH, HALF)
    pooled = _sc_pool(x3, table)
    return _tc_head(pooled, W.T, b.reshape(1, OUTPUT_DIM))


# speedup vs baseline: 1.8810x; 1.8810x over previous
import functools

import jax
import jax.numpy as jnp
from jax import lax
from jax.experimental import pallas as pl
from jax.experimental.pallas import tpu as pltpu
from jax.experimental.pallas import tpu_sc as plsc

BATCH = 4096
HIST = 200
EMBED_DIM = 64
OUTPUT_DIM = 128

NC = 2
NS = 16
NW = NC * NS

ROWS_PER_W = BATCH // NW
CW = 128
CHUNKS = ROWS_PER_W * HIST // CW
PERIOD = 25
ROWS_PER_PERIOD = PERIOD * CW // HIST
NBUF = 5


def _sc_pool_body(x_hbm, table_hbm, out_hbm, idx_v, ring_v, out_stage, *sems):
    wid = lax.axis_index("s") * NC + lax.axis_index("c")
    pltpu.sync_copy(x_hbm.at[pl.ds(wid * CHUNKS, CHUNKS)], idx_v)

    def issue(s, c):
        pltpu.make_async_copy(
            table_hbm.at[idx_v.at[c]],
            ring_v.at[s],
            sems[s],
        ).start()

    for s in range(NBUF):
        issue(s, s)

    scale = jnp.float32(1.0 / HIST)
    z = jnp.zeros((16,), jnp.float32)

    def chunk_sum(s, lo, hi, a):
        def body(i, a):
            a0, a1, a2, a3 = a
            a0 = a0 + ring_v[s, i, pl.ds(0, 16)]
            a1 = a1 + ring_v[s, i, pl.ds(16, 16)]
            a2 = a2 + ring_v[s, i, pl.ds(32, 16)]
            a3 = a3 + ring_v[s, i, pl.ds(48, 16)]
            return (a0, a1, a2, a3)

        if lo >= hi:
            return a
        return lax.fori_loop(lo, hi, body, a)

    def emit(m, a):
        a0, a1, a2, a3 = a
        out_stage[m, pl.ds(0, 16)] = a0 * scale
        out_stage[m, pl.ds(16, 16)] = a1 * scale
        out_stage[m, pl.ds(32, 16)] = a2 * scale
        out_stage[m, pl.ds(48, 16)] = a3 * scale

    def period(g, carry):
        acc = (z, z, z, z)
        for j in range(PERIOD):
            s = j % NBUF
            c = g * PERIOD + j
            pltpu.make_async_copy(
                table_hbm.at[idx_v.at[c]],
                ring_v.at[s],
                sems[s],
            ).wait()

            b0 = (j * CW) // HIST
            bound = HIST * (b0 + 1) - j * CW
            if bound <= CW:
                acc = chunk_sum(s, 0, bound, acc)
                emit(g * ROWS_PER_PERIOD + b0, acc)
                acc = chunk_sum(s, bound, CW, (z, z, z, z))
            else:
                acc = chunk_sum(s, 0, CW, acc)

            @pl.when(c + NBUF < CHUNKS)
            def _():
                issue(s, c + NBUF)

        return carry

    lax.fori_loop(0, CHUNKS // PERIOD, period, 0)
    pltpu.sync_copy(out_stage, out_hbm.at[pl.ds(wid * ROWS_PER_W, ROWS_PER_W)])


_sc_pool = functools.partial(
    pl.kernel,
    out_type=jax.ShapeDtypeStruct((BATCH, EMBED_DIM), jnp.float32),
    mesh=plsc.VectorSubcoreMesh(core_axis_name="c", subcore_axis_name="s"),
    scratch_types=[
        pltpu.VMEM((CHUNKS, CW), jnp.int32),
        pltpu.VMEM((NBUF, CW, EMBED_DIM), jnp.float32),
        pltpu.VMEM((ROWS_PER_W, EMBED_DIM), jnp.float32),
    ]
    + [pltpu.SemaphoreType.DMA] * NBUF,
    compiler_params=pltpu.CompilerParams(use_tc_tiling_on_sc=False),
)(_sc_pool_body)


def _tc_head_body(p_ref, w_ref, b_ref, o_ref):
    out = jnp.dot(p_ref[...], w_ref[...], preferred_element_type=jnp.float32)
    out = out + b_ref[...]
    ss = jnp.sum(out * out, axis=1, keepdims=True)
    norm = jnp.sqrt(ss)
    o_ref[...] = out / jnp.maximum(norm, 1e-12)


def _tc_head(pooled, wt, b2):
    blk = 512
    return pl.pallas_call(
        _tc_head_body,
        grid=(BATCH // blk,),
        in_specs=[
            pl.BlockSpec((blk, EMBED_DIM), lambda i: (i, 0)),
            pl.BlockSpec((EMBED_DIM, OUTPUT_DIM), lambda i: (0, 0)),
            pl.BlockSpec((1, OUTPUT_DIM), lambda i: (0, 0)),
        ],
        out_specs=pl.BlockSpec((blk, OUTPUT_DIM), lambda i: (i, 0)),
        out_shape=jax.ShapeDtypeStruct((BATCH, OUTPUT_DIM), jnp.float32),
    )(pooled, wt, b2)


def kernel(x, table, W, b):
    x2 = x.astype(jnp.int32).reshape(BATCH * HIST // CW, CW)
    pooled = _sc_pool(x2, table)
    return _tc_head(pooled, W.T, b.reshape(1, OUTPUT_DIM))
